# baseline (speedup 1.0000x reference)
import jax
import jax.numpy as jnp
from jax.experimental import pallas as pl
from jax.experimental.pallas import tpu as pltpu

_B, _C, _T = 64, 64, 80
_EPS = 1e-9
_WB = 128
_NB = (_B * _T) // _WB
_G = _C // 8


def _div_kernel(x_ref, y_ref, out_ref, xt_ref, yt_ref):
    for b in range(_B):
        xt_ref[:, b * _T:(b + 1) * _T] = x_ref[b]
        yt_ref[:, b * _T:(b + 1) * _T] = y_ref[b]

    def body(c, accs):
        off = pl.multiple_of(c * _WB, _WB)
        xg = [xt_ref[pl.ds(8 * g, 8), pl.ds(off, _WB)] for g in range(_G)]
        yg = [yt_ref[pl.ds(8 * g, 8), pl.ds(off, _WB)] for g in range(_G)]
        new = list(accs)
        group = 0
        for a in range(5):
            if a == 0:
                lx, ly = xg, yg
            else:
                lx = [pltpu.roll(v, 8 - a, 0) for v in xg]
                ly = [pltpu.roll(v, 8 - a, 0) for v in yg]
            for b in range(_G):
                ds = []
                for g in range(_G):
                    delta = 8 * a - b if g < _G - b else 8 * a + 8 - b
                    if not 1 <= delta <= _C // 2:
                        continue
                    j = (g + b) % _G
                    dx = lx[g] - xg[j]
                    dy = ly[g] - yg[j]
                    s = dx * dx + dy * dy + _EPS
                    d = s * jax.lax.rsqrt(s)
                    if delta == _C // 2:
                        d = 0.5 * d
                    ds.append(d)
                while len(ds) > 1:
                    ds = [p + q for p, q in zip(ds[::2], ds[1::2])] + (
                        [ds[-1]] if len(ds) % 2 else [])
                if ds:
                    new[group] = new[group] + ds[0]
                    group ^= 1
        return tuple(new)

    zero = jnp.zeros((8, _WB), jnp.float32)
    accs = jax.lax.fori_loop(0, _NB, body, (zero, zero), unroll=2)
    total = 2.0 * sum(jnp.sum(a) for a in accs)
    scale = -1.0 / (_T * _C * (_C - 1) * _B)
    out_ref[...] = jnp.reshape(total * scale, (1, 1))


def kernel(predicted_trajectory):
    traj = predicted_trajectory[..., :2]
    x = traj[..., 0]
    y = traj[..., 1]
    out = pl.pallas_call(
        _div_kernel,
        out_shape=jax.ShapeDtypeStruct((1, 1), jnp.float32),
        scratch_shapes=[
            pltpu.VMEM((_C, _B * _T), jnp.float32),
            pltpu.VMEM((_C, _B * _T), jnp.float32),
        ],
    )(x, y)
    return out.reshape(())


# speedup vs baseline: 1.0070x; 1.0070x over previous
import jax
import jax.numpy as jnp
from jax.experimental import pallas as pl
from jax.experimental.pallas import tpu as pltpu

_B, _C, _T = 64, 64, 80
_EPS = 1e-9
_WB = 128
_NB = (_B * _T) // _WB
_G = _C // 8


def _div_kernel(x_ref, y_ref, out_ref, xt_ref, yt_ref):
    for b in range(_B):
        xt_ref[:, b * _T:(b + 1) * _T] = x_ref[b]
        yt_ref[:, b * _T:(b + 1) * _T] = y_ref[b]

    def body(c, accs):
        off = pl.multiple_of(c * _WB, _WB)
        xg = [xt_ref[pl.ds(8 * g, 8), pl.ds(off, _WB)] for g in range(_G)]
        yg = [yt_ref[pl.ds(8 * g, 8), pl.ds(off, _WB)] for g in range(_G)]
        new = list(accs)
        group = 0
        for a in range(5):
            if a == 0:
                lx, ly = xg, yg
            else:
                lx = [pltpu.roll(v, 8 - a, 0) for v in xg]
                ly = [pltpu.roll(v, 8 - a, 0) for v in yg]
            for b in range(_G):
                ds = []
                for g in range(_G):
                    delta = 8 * a - b if g < _G - b else 8 * a + 8 - b
                    if not 1 <= delta <= _C // 2:
                        continue
                    j = (g + b) % _G
                    dx = lx[g] - xg[j]
                    dy = ly[g] - yg[j]
                    s = dx * dx + dy * dy + _EPS
                    d = s * jax.lax.rsqrt(s)
                    if delta == _C // 2:
                        d = 0.5 * d
                    ds.append(d)
                while len(ds) > 1:
                    ds = [p + q for p, q in zip(ds[::2], ds[1::2])] + (
                        [ds[-1]] if len(ds) % 2 else [])
                if ds:
                    new[group] = new[group] + ds[0]
                    group ^= 1
        return tuple(new)

    zero = jnp.zeros((8, _WB), jnp.float32)
    accs = jax.lax.fori_loop(0, _NB, body, (zero, zero), unroll=4)
    total = 2.0 * sum(jnp.sum(a) for a in accs)
    scale = -1.0 / (_T * _C * (_C - 1) * _B)
    out_ref[...] = jnp.reshape(total * scale, (1, 1))


def kernel(predicted_trajectory):
    traj = predicted_trajectory[..., :2]
    x = traj[..., 0]
    y = traj[..., 1]
    out = pl.pallas_call(
        _div_kernel,
        out_shape=jax.ShapeDtypeStruct((1, 1), jnp.float32),
        scratch_shapes=[
            pltpu.VMEM((_C, _B * _T), jnp.float32),
            pltpu.VMEM((_C, _B * _T), jnp.float32),
        ],
    )(x, y)
    return out.reshape(())
